# baseline (speedup 1.0000x reference)
import functools

import jax
import jax.numpy as jnp
from jax import lax
from jax.experimental import pallas as pl
from jax.experimental.pallas import tpu as pltpu
from jax.experimental.pallas import tpu_sc as plsc

VOCAB = 100000
BATCH = 32
Q_LEN = 8
ROWS = BATCH * Q_LEN

CHUNK = 8192
N_FULL = VOCAB // CHUNK
TAIL = VOCAB - N_FULL * CHUNK

def _prelude(ls_rep_ref, ls_sq_ref, ids_sq_ref, base_ref, peaks_ref, pos_ref):
    a_rep = jax.nn.sigmoid(ls_rep_ref[...])
    base_ref[...] = jnp.maximum(jnp.log((1.0 - a_rep) * jnp.float32(1.0 / VOCAB)), jnp.float32(-1e6))
    a_sq = jax.nn.sigmoid(ls_sq_ref[...])
    b_sq = (1.0 - a_sq) * jnp.float32(1.0 / VOCAB)
    peaks_ref[...] = jnp.maximum(jnp.log(b_sq + a_sq), jnp.float32(-1e6))
    r = (
        lax.broadcasted_iota(jnp.int32, (16, 16), 0) * 16
        + lax.broadcasted_iota(jnp.int32, (16, 16), 1)
    )
    pos_ref[...] = r * VOCAB + ids_sq_ref[...]


def _sc_body(base_hbm, peaks_hbm, pos_hbm, out_hbm, *rest):
    bufs = rest[:8]
    val16, peaks_v, pos_v, sem, sem2 = rest[8:]
    info = plsc.get_sparse_core_info()
    nc = info.num_cores
    wid = lax.axis_index("s") * nc + lax.axis_index("c")
    rows_per_w = ROWS // (nc * info.num_subcores)
    pltpu.sync_copy(peaks_hbm.at[wid // 2, pl.ds((wid % 2) * 8, 8)], peaks_v)
    pltpu.sync_copy(pos_hbm.at[wid // 2, pl.ds((wid % 2) * 8, 8)], pos_v)

    descs = []
    for r in range(rows_per_w):
        row = wid * rows_per_w + r
        pltpu.sync_copy(base_hbm.at[row], val16)
        v = val16[...]

        buf = bufs[r]

        def fill(i, _, buf=buf, v=v):
            base = i * 256
            for k in range(16):
                buf[pl.ds(base + k * 16, 16)] = v
            return 0

        lax.fori_loop(0, CHUNK // 256, fill, 0)
        off = row * VOCAB
        for c in range(N_FULL):
            descs.append(
                pltpu.async_copy(buf, out_hbm.at[pl.ds(off + c * CHUNK, CHUNK)], sem)
            )
        descs.append(
            pltpu.async_copy(buf, out_hbm.at[pl.ds(off + VOCAB - CHUNK, CHUNK)], sem)
        )
    for d in descs:
        d.wait()
    pltpu.async_copy(peaks_v, out_hbm.at[pos_v], sem2).wait()


@jax.jit
def kernel(log_snr, input_ids):
    ls_flat = log_snr.reshape(ROWS)
    ls_rep = jnp.broadcast_to(ls_flat[:, None], (ROWS, 16))
    ls_sq = ls_flat.reshape(16, 16)
    ids_sq = input_ids.astype(jnp.int32).reshape(16, 16)

    base_rep, peaks_sq, pos_sq = pl.pallas_call(
        _prelude,
        out_shape=[
            jax.ShapeDtypeStruct((ROWS, 16), jnp.float32),
            jax.ShapeDtypeStruct((16, 16), jnp.float32),
            jax.ShapeDtypeStruct((16, 16), jnp.int32),
        ],
    )(ls_rep, ls_sq, ids_sq)

    mesh = plsc.VectorSubcoreMesh(core_axis_name="c", subcore_axis_name="s")
    sc_fill = functools.partial(
        pl.kernel,
        out_type=jax.ShapeDtypeStruct((ROWS * VOCAB,), jnp.float32),
        mesh=mesh,
        scratch_types=[
            *[pltpu.VMEM((CHUNK,), jnp.float32) for _ in range(8)],
            pltpu.VMEM((16,), jnp.float32),
            pltpu.VMEM((8,), jnp.float32),
            pltpu.VMEM((8,), jnp.int32),
            pltpu.SemaphoreType.DMA,
            pltpu.SemaphoreType.DMA,
        ],
    )(_sc_body)
    flat = sc_fill(base_rep, peaks_sq, pos_sq)
    return flat.reshape(BATCH, Q_LEN, VOCAB)
